# Initial kernel scaffold; baseline (speedup 1.0000x reference)
import jax
import jax.numpy as jnp
from jax.experimental import pallas as pl


def kernel(z, edge_index, W1, b1, W2, b2, U, V, F, Wd1, bd1, Wd2, bd2):
    raise NotImplementedError("write your pallas kernel here")


# speedup vs baseline: 13.9961x; 13.9961x over previous
import functools

import jax
import jax.numpy as jnp
from jax import lax
from jax.experimental import pallas as pl
from jax.experimental.pallas import tpu as pltpu
from jax.experimental.pallas import tpu_sc as plsc

S_N = 10000
C_T = 4096
G_ST = 128
G_SC = 2000
D_2 = 32
K_SP = 20
H_1 = 64
DEC_W = 256
E_N = 320000

NC = 2
NS = 16
LN = 128
N_CHUNKS = 2560
E_PAD = N_CHUNKS * LN
CPT = N_CHUNKS // (NC * NS)
RPT = 624
S_PAD = S_N + 8


def _make_deg_kernel():
    mesh = plsc.VectorSubcoreMesh(core_axis_name="c", subcore_axis_name="s")

    @functools.partial(
        pl.kernel,
        mesh=mesh,
        compiler_params=pltpu.CompilerParams(use_tc_tiling_on_sc=False),
        out_type=jax.ShapeDtypeStruct((NC, S_N, 16), jnp.float32),
        scratch_types=[
            pltpu.VMEM((CPT, LN), jnp.int32),
            pltpu.VMEM((LN, 16), jnp.float32),
            pltpu.VMEM_SHARED((S_PAD, 16), jnp.float32),
        ],
    )
    def k(ones_hbm, dstc_hbm, zeros_hbm, out_hbm, dst_v, ones_v, acc):
        c = lax.axis_index("c")
        s = lax.axis_index("s")
        wid = c * NS + s
        pltpu.sync_copy(dstc_hbm.at[pl.ds(wid * CPT, CPT), :], dst_v)
        pltpu.sync_copy(ones_hbm, ones_v)
        pltpu.sync_copy(zeros_hbm, acc.at[pl.ds(s * RPT, RPT), :])

        @pl.when(s == NS - 1)
        def _():
            pltpu.sync_copy(zeros_hbm.at[pl.ds(0, 16), :],
                            acc.at[pl.ds(NS * RPT, 16), :])
            pltpu.sync_copy(zeros_hbm.at[pl.ds(0, 8), :], acc.at[pl.ds(S_N, 8), :])

        plsc.subcore_barrier()

        def step(j, carry):
            pltpu.sync_copy(ones_v, acc.at[dst_v.at[j]], add=True)
            return carry

        lax.fori_loop(0, CPT, step, 0)
        plsc.subcore_barrier()
        pltpu.sync_copy(acc.at[pl.ds(s * RPT, RPT), :],
                        out_hbm.at[c, pl.ds(s * RPT, RPT), :])

        @pl.when(s == NS - 1)
        def _():
            pltpu.sync_copy(acc.at[pl.ds(NS * RPT, 16), :],
                            out_hbm.at[c, pl.ds(NS * RPT, 16), :])

    return k


def _make_scatter_kernel(d):
    mesh = plsc.VectorSubcoreMesh(core_axis_name="c", subcore_axis_name="s")

    @functools.partial(
        pl.kernel,
        mesh=mesh,
        compiler_params=pltpu.CompilerParams(use_tc_tiling_on_sc=False),
        out_type=jax.ShapeDtypeStruct((NC, S_N, d), jnp.float32),
        scratch_types=[
            pltpu.VMEM((CPT, LN), jnp.int32),
            pltpu.VMEM((CPT, LN), jnp.int32),
            pltpu.VMEM((LN, d), jnp.float32),
            pltpu.VMEM_SHARED((S_PAD, d), jnp.float32),
            pltpu.SemaphoreType.DMA,
        ],
    )
    def k(y_hbm, srcc_hbm, dstc_hbm, zeros_hbm, out_hbm,
          src_v, dst_v, rows_v, acc, sem):
        c = lax.axis_index("c")
        s = lax.axis_index("s")
        wid = c * NS + s
        pltpu.sync_copy(srcc_hbm.at[pl.ds(wid * CPT, CPT), :], src_v)
        pltpu.sync_copy(dstc_hbm.at[pl.ds(wid * CPT, CPT), :], dst_v)
        pltpu.sync_copy(zeros_hbm, acc.at[pl.ds(s * RPT, RPT), :])

        @pl.when(s == NS - 1)
        def _():
            pltpu.sync_copy(zeros_hbm.at[pl.ds(0, 16), :],
                            acc.at[pl.ds(NS * RPT, 16), :])
            pltpu.sync_copy(zeros_hbm.at[pl.ds(0, 8), :], acc.at[pl.ds(S_N, 8), :])

        plsc.subcore_barrier()

        def step(j, carry):
            pltpu.async_copy(y_hbm.at[src_v.at[j]], rows_v, sem).wait()
            pltpu.sync_copy(rows_v, acc.at[dst_v.at[j]], add=True)
            return carry

        lax.fori_loop(0, CPT, step, 0)
        plsc.subcore_barrier()
        pltpu.sync_copy(acc.at[pl.ds(s * RPT, RPT), :],
                        out_hbm.at[c, pl.ds(s * RPT, RPT), :])

        @pl.when(s == NS - 1)
        def _():
            pltpu.sync_copy(acc.at[pl.ds(NS * RPT, 16), :],
                            out_hbm.at[c, pl.ds(NS * RPT, 16), :])

    return k


_deg_kernel = _make_deg_kernel()
_scat64 = _make_scatter_kernel(H_1)
_scat32 = _make_scatter_kernel(D_2)


def _tc1_body(dp0, dp1, z, w1, dis_o, y1_o):
    deg = dp0[:, 0:1] + dp1[:, 0:1] + 1.0
    dis = lax.rsqrt(deg)
    dis_o[...] = dis
    y1_o[...] = jnp.dot(z[...], w1[...], preferred_element_type=jnp.float32) * dis


def _tc2_body(p0, p1, y1, dis_r, b1, w2, y2_o):
    dis = dis_r[...]
    h = jnp.maximum((p0[...] + p1[...] + y1[...]) * dis + b1[...], 0.0)
    y2_o[...] = jnp.dot(h, w2[...], preferred_element_type=jnp.float32) * dis


def _tc3_body(u, q0, q1, y2, dis_r, b2, a_o, h_o):
    x = u[...]
    m = jnp.max(x, axis=1, keepdims=True)
    e = jnp.exp(x - m)
    ssum = jnp.sum(e, axis=1, keepdims=True)
    a_o[...] = e / ssum
    h_o[...] = (q0[...] + q1[...] + y2[...]) * dis_r[...] + b2[...]


def _tc4_body(v, f, wd1, bd1, wd2, bd2, b_o, m_o):
    x = v[...]
    m = jnp.max(x, axis=1, keepdims=True)
    e = jnp.exp(x - m)
    bmat = e / jnp.sum(e, axis=1, keepdims=True)
    b_o[...] = bmat
    srow = jnp.sum(bmat, axis=0, keepdims=True)
    rows = lax.broadcasted_iota(jnp.int32, (K_SP, K_SP), 0)
    cols = lax.broadcasted_iota(jnp.int32, (K_SP, K_SP), 1)
    diag = jnp.where(rows == cols, srow, 0.0)
    p1 = jnp.maximum(
        jnp.dot(f[...], wd1[...], preferred_element_type=jnp.float32) + bd1[...], 0.0)
    p2 = jnp.dot(p1, wd2[...], preferred_element_type=jnp.float32) + bd2[...]
    m_o[...] = jnp.dot(diag, p2, preferred_element_type=jnp.float32)


_RB = 400


def kernel(z, edge_index, W1, b1, W2, b2, U, V, F, Wd1, bd1, Wd2, bd2):
    src = edge_index[0]
    dst = edge_index[1]
    pad = E_PAD - E_N
    pad_dst = S_N + (jnp.arange(pad, dtype=jnp.int32) % 8)
    srcc = jnp.concatenate([src, jnp.zeros((pad,), jnp.int32)]).reshape(N_CHUNKS, LN)
    dstc = jnp.concatenate([dst, pad_dst]).reshape(N_CHUNKS, LN)

    ones16 = jnp.ones((LN, 16), jnp.float32)
    zeros16 = jnp.zeros((RPT, 16), jnp.float32)
    zeros64 = jnp.zeros((RPT, H_1), jnp.float32)
    zeros32 = jnp.zeros((RPT, D_2), jnp.float32)

    deg_parts = _deg_kernel(ones16, dstc, zeros16)

    dis, y1 = pl.pallas_call(
        _tc1_body,
        out_shape=(
            jax.ShapeDtypeStruct((S_N, 1), jnp.float32),
            jax.ShapeDtypeStruct((S_N, H_1), jnp.float32),
        ),
    )(deg_parts[0], deg_parts[1], z, W1)

    p1 = _scat64(y1, srcc, dstc, zeros64)

    y2 = pl.pallas_call(
        _tc2_body,
        out_shape=jax.ShapeDtypeStruct((S_N, D_2), jnp.float32),
    )(p1[0], p1[1], y1, dis, b1.reshape(1, H_1), W2)

    p2 = _scat32(y2, srcc, dstc, zeros32)

    grid = S_N // _RB
    A, h = pl.pallas_call(
        _tc3_body,
        grid=(grid,),
        in_specs=[
            pl.BlockSpec((_RB, C_T), lambda i: (i, 0)),
            pl.BlockSpec((_RB, D_2), lambda i: (i, 0)),
            pl.BlockSpec((_RB, D_2), lambda i: (i, 0)),
            pl.BlockSpec((_RB, D_2), lambda i: (i, 0)),
            pl.BlockSpec((_RB, 1), lambda i: (i, 0)),
            pl.BlockSpec((1, D_2), lambda i: (0, 0)),
        ],
        out_specs=[
            pl.BlockSpec((_RB, C_T), lambda i: (i, 0)),
            pl.BlockSpec((_RB, D_2), lambda i: (i, 0)),
        ],
        out_shape=(
            jax.ShapeDtypeStruct((S_N, C_T), jnp.float32),
            jax.ShapeDtypeStruct((S_N, D_2), jnp.float32),
        ),
    )(U, p2[0], p2[1], y2, dis, b2.reshape(1, D_2))

    B, M_rec = pl.pallas_call(
        _tc4_body,
        out_shape=(
            jax.ShapeDtypeStruct((C_T, K_SP), jnp.float32),
            jax.ShapeDtypeStruct((K_SP, G_SC), jnp.float32),
        ),
    )(V, F, Wd1, bd1.reshape(1, DEC_W), Wd2, bd2.reshape(1, G_SC))

    return (A, B, h, M_rec, F)
